# baseline (speedup 1.0000x reference)
import functools

import jax
import jax.numpy as jnp
from jax import lax
from jax.experimental import pallas as pl
from jax.experimental.pallas import tpu as pltpu

H = 16
HD = 64
LR = 0.1
C = 32
F32 = jnp.float32


BF16 = jnp.bfloat16


def _mm_body(x_ref, w_ref, o_ref):
    o_ref[...] = jnp.dot(x_ref[...], w_ref[...],
                         preferred_element_type=F32).astype(o_ref.dtype)


def _matmul(x, w, out_dtype, bm=1024, bn=1024):
    m, k = x.shape
    _, n = w.shape
    return pl.pallas_call(
        _mm_body,
        grid=(m // bm, n // bn),
        in_specs=[
            pl.BlockSpec((bm, k), lambda i, j: (i, 0)),
            pl.BlockSpec((k, bn), lambda i, j: (0, j)),
        ],
        out_specs=pl.BlockSpec((bm, bn), lambda i, j: (i, j)),
        out_shape=jax.ShapeDtypeStruct((m, n), out_dtype),
        compiler_params=pltpu.CompilerParams(
            dimension_semantics=("parallel", "parallel")),
        name="proj_mm",
    )(x, w)


def _solve_body(k_ref, v_ref, wx_ref, *, r, beta, cg_n):
    r2 = 2 * r
    rows = lax.broadcasted_iota(jnp.int32, (r, r2), 0)
    cols = lax.broadcasted_iota(jnp.int32, (r, r2), 1)
    colm = cols & (r - 1)
    strict_p = (colm >> 2) < (rows >> 2)
    ident2 = jnp.where(colm == rows, 1.0, 0.0).astype(BF16)
    lane_lo = cols < r
    lane_even64 = ((cols >> 6) & 1) == 0
    sb = float(beta) ** 0.5

    def bdiag(tp):
        top = jnp.where(lane_lo, tp, jnp.bfloat16(0))
        bot = jnp.where(lane_lo, jnp.bfloat16(0), tp)
        return jnp.concatenate([top, bot], axis=0)

    for ci in range(cg_n):
        v12 = v_ref[ci * r:(ci + 1) * r, :]
        k12 = k_ref[ci * r:(ci + 1) * r, :]
        k1s = k12[:, :HD] * sb
        k2s = k12[:, HD:] * sb
        g1 = lax.dot_general(k1s, k1s, (((1,), (1,)), ((), ())),
                             preferred_element_type=F32)
        g2 = lax.dot_general(k2s, k2s, (((1,), (1,)), ((), ())),
                             preferred_element_type=F32)
        gp = jnp.concatenate([g1, g2], axis=1)
        ad = bdiag(jnp.where(strict_p, gp, 0.0).astype(BF16))
        vkcat = jnp.concatenate([v12, k12], axis=1)
        rhs = jnp.concatenate([
            jnp.where(lane_even64, vkcat, jnp.bfloat16(0)),
            jnp.where(lane_even64, jnp.bfloat16(0), vkcat),
        ], axis=0)
        t = ident2
        gsz, sh = 4, 2
        while gsz < r:
            td = bdiag(t)
            u = jnp.dot(t, ad, preferred_element_type=F32)
            u = jnp.dot(u.astype(BF16), td, preferred_element_type=F32)
            rg = rows >> sh
            cg = colm >> sh
            msub = ((rg & 1) == 1) & (cg == rg - 1)
            t = t - jnp.where(msub, u.astype(BF16), jnp.bfloat16(0))
            gsz, sh = gsz * 2, sh + 1
        wx_ref[ci * r:(ci + 1) * r, :] = jnp.dot(
            t, rhs, preferred_element_type=F32).astype(BF16)


def _sweep_body(wx0, wx1, wx2, wx3, q_ref, k_ref, o_ref, n_ref,
                *, r, beta, hpc, cb_n):
    c = pl.program_id(1)

    @pl.when(c == 0)
    def _():
        n_ref[...] = jnp.zeros_like(n_ref)

    rows = lax.broadcasted_iota(jnp.int32, (r, r), 0)
    cols = lax.broadcasted_iota(jnp.int32, (r, r), 1)
    incl = (cols >> 2) <= (rows >> 2)
    wxr = (wx0, wx1, wx2, wx3)

    for cc in range(cb_n):
        rs = slice(cc * r, (cc + 1) * r)
        for j in range(hpc):
            pr, odd = j >> 1, j & 1
            w = wxr[pr][rs, odd * HD:(odd + 1) * HD]
            x = wxr[pr][rs, 2 * HD + odd * HD:2 * HD + (odd + 1) * HD]
            q = q_ref[rs, j * HD:(j + 1) * HD]
            kk = k_ref[rs, j * HD:(j + 1) * HD]
            n = n_ref[j]
            nb = n.astype(BF16)
            e = w.astype(F32) - jnp.dot(x, nb, preferred_element_type=F32)
            eb = e.astype(BF16)
            qk = lax.dot_general(q, kk, (((1,), (1,)), ((), ())),
                                 preferred_element_type=F32)
            aq = jnp.where(incl, beta * qk, 0.0).astype(BF16)
            o_ref[rs, j * HD:(j + 1) * HD] = (
                jnp.dot(q, nb, preferred_element_type=F32)
                + jnp.dot(aq, eb, preferred_element_type=F32)).astype(BF16)
            n_ref[j] = n + beta * lax.dot_general(
                kk, eb, (((0,), (0,)), ((), ())),
                preferred_element_type=F32)


def kernel(x, Wq, Wk, Wv, Wo):
    b, s, d = x.shape
    r = C * b
    nc = s // C
    beta = LR / b
    hpc = H // 2

    xt = x.transpose(1, 0, 2).reshape(s * b, d).astype(BF16)
    wqkv = jnp.concatenate([Wq.T, Wk.T, Wv.T], axis=1).astype(BF16)

    qkv = _matmul(xt, wqkv, BF16, bm=1024, bn=1024)

    cg_n = 8
    cb_n = 4
    solve = functools.partial(_solve_body, r=r, beta=beta, cg_n=cg_n)
    wx = pl.pallas_call(
        solve,
        grid=(H // 2, nc // cg_n),
        in_specs=[
            pl.BlockSpec((cg_n * r, 2 * HD),
                         lambda p, c: (c, H // 2 + p)),
            pl.BlockSpec((cg_n * r, 2 * HD),
                         lambda p, c: (c, H + p)),
        ],
        out_specs=pl.BlockSpec((cg_n * r, 4 * HD),
                               lambda p, c: (p * (nc // cg_n) + c, 0)),
        out_shape=jax.ShapeDtypeStruct((nc * (H // 2) * r, 4 * HD), BF16),
        compiler_params=pltpu.CompilerParams(
            dimension_semantics=("parallel", "parallel")),
        name="chunk_solve",
    )(qkv, qkv)

    sweep = functools.partial(_sweep_body, r=r, beta=beta, hpc=hpc, cb_n=cb_n)
    nb_c = nc // cb_n
    wx_spec = [
        pl.BlockSpec((cb_n * r, 4 * HD),
                     functools.partial(
                         lambda i, gg, c: ((gg * 4 + i) * nb_c + c, 0), i))
        for i in range(4)
    ]
    o = pl.pallas_call(
        sweep,
        grid=(2, nb_c),
        in_specs=wx_spec + [
            pl.BlockSpec((cb_n * r, hpc * HD), lambda gg, c: (c, gg)),
            pl.BlockSpec((cb_n * r, hpc * HD), lambda gg, c: (c, 2 + gg)),
        ],
        out_specs=pl.BlockSpec((cb_n * r, hpc * HD), lambda gg, c: (c, gg)),
        out_shape=jax.ShapeDtypeStruct((s * b, d), BF16),
        scratch_shapes=[pltpu.VMEM((hpc, HD, HD), F32)],
        compiler_params=pltpu.CompilerParams(
            dimension_semantics=("parallel", "arbitrary")),
        name="chunk_sweep",
    )(wx, wx, wx, wx, qkv, qkv)

    out = _matmul(o, Wo.T.astype(BF16), F32, bm=1024, bn=1024)
    return out.reshape(s, b, d).transpose(1, 0, 2)


# speedup vs baseline: 1.3012x; 1.3012x over previous
import functools

import jax
import jax.numpy as jnp
from jax import lax
from jax.experimental import pallas as pl
from jax.experimental.pallas import tpu as pltpu

H = 16
HD = 64
LR = 0.1
C = 32
F32 = jnp.float32


BF16 = jnp.bfloat16


def _mm_body(x_ref, w_ref, o_ref):
    o_ref[...] = jnp.dot(x_ref[...], w_ref[...],
                         preferred_element_type=F32).astype(o_ref.dtype)


def _matmul(x, w, out_dtype, bm=1024, bn=1024):
    m, k = x.shape
    _, n = w.shape
    return pl.pallas_call(
        _mm_body,
        grid=(m // bm, n // bn),
        in_specs=[
            pl.BlockSpec((bm, k), lambda i, j: (i, 0)),
            pl.BlockSpec((k, bn), lambda i, j: (0, j)),
        ],
        out_specs=pl.BlockSpec((bm, bn), lambda i, j: (i, j)),
        out_shape=jax.ShapeDtypeStruct((m, n), out_dtype),
        compiler_params=pltpu.CompilerParams(
            dimension_semantics=("parallel", "parallel")),
        name="proj_mm",
    )(x, w)


def _solve_body(k_ref, v_ref, wx_ref, *, r, beta, cg_n):
    r2 = 2 * r
    rows = lax.broadcasted_iota(jnp.int32, (r, r2), 0)
    cols = lax.broadcasted_iota(jnp.int32, (r, r2), 1)
    colm = cols & (r - 1)
    strict_p = (colm >> 2) < (rows >> 2)
    ident2 = jnp.where(colm == rows, 1.0, 0.0).astype(BF16)
    lane_lo = cols < r
    lane_even64 = ((cols >> 6) & 1) == 0
    sb = float(beta) ** 0.5

    def bdiag(tp):
        top = jnp.where(lane_lo, tp, jnp.bfloat16(0))
        bot = jnp.where(lane_lo, jnp.bfloat16(0), tp)
        return jnp.concatenate([top, bot], axis=0)

    for ci in range(cg_n):
        v12 = v_ref[ci * r:(ci + 1) * r, :]
        k12 = k_ref[ci * r:(ci + 1) * r, :]
        k1s = k12[:, :HD] * sb
        k2s = k12[:, HD:] * sb
        g1 = lax.dot_general(k1s, k1s, (((1,), (1,)), ((), ())),
                             preferred_element_type=F32)
        g2 = lax.dot_general(k2s, k2s, (((1,), (1,)), ((), ())),
                             preferred_element_type=F32)
        gp = jnp.concatenate([g1, g2], axis=1)
        ad = bdiag(jnp.where(strict_p, gp, 0.0).astype(BF16))
        vkcat = jnp.concatenate([v12, k12], axis=1)
        rhs = jnp.concatenate([
            jnp.where(lane_even64, vkcat, jnp.bfloat16(0)),
            jnp.where(lane_even64, jnp.bfloat16(0), vkcat),
        ], axis=0)
        m4 = (((rows >> 2) & 1) == 1) & ((colm >> 2) == (rows >> 2) - 1)
        t = ident2 - jnp.where(m4, gp, 0.0).astype(BF16)
        gsz, sh = 8, 3
        while gsz < r:
            td = bdiag(t)
            u = jnp.dot(t, ad, preferred_element_type=F32)
            u = jnp.dot(u.astype(BF16), td, preferred_element_type=F32)
            rg = rows >> sh
            cg = colm >> sh
            msub = ((rg & 1) == 1) & (cg == rg - 1)
            t = t - jnp.where(msub, u.astype(BF16), jnp.bfloat16(0))
            gsz, sh = gsz * 2, sh + 1
        wx_ref[ci * r:(ci + 1) * r, :] = jnp.dot(
            t, rhs, preferred_element_type=F32).astype(BF16)


def _sweep_body(wx0, wx1, wx2, wx3, q_ref, k_ref, o_ref, n_ref,
                *, r, beta, hpc, cb_n):
    c = pl.program_id(1)

    @pl.when(c == 0)
    def _():
        n_ref[...] = jnp.zeros_like(n_ref)

    rows = lax.broadcasted_iota(jnp.int32, (r, r), 0)
    cols = lax.broadcasted_iota(jnp.int32, (r, r), 1)
    incl = (cols >> 2) <= (rows >> 2)
    wxr = (wx0, wx1, wx2, wx3)
    q4 = 4 * HD

    def bdiag4(p4):
        lane = lax.broadcasted_iota(jnp.int32, p4.shape, 1) >> 6
        return jnp.concatenate(
            [jnp.where(lane == i, p4, jnp.bfloat16(0)) for i in range(4)],
            axis=0)

    for cc in range(cb_n):
        rs = slice(cc * r, (cc + 1) * r)
        for grp in range(hpc // 4):
            pr0 = grp * 2
            w4 = jnp.concatenate(
                [wxr[pr0][rs, 0:2 * HD], wxr[pr0 + 1][rs, 0:2 * HD]], axis=1)
            x4 = jnp.concatenate(
                [wxr[pr0][rs, 2 * HD:4 * HD],
                 wxr[pr0 + 1][rs, 2 * HD:4 * HD]], axis=1)
            qq = q_ref[rs, grp * q4:(grp + 1) * q4]
            n4 = n_ref[grp]
            nd = bdiag4(n4.astype(BF16))
            xq = jnp.concatenate([x4, qq], axis=0)
            xqn = jnp.dot(xq, nd, preferred_element_type=F32)
            e4 = w4.astype(F32) - xqn[:r]
            eb4 = e4.astype(BF16)
            ed = bdiag4(eb4)
            aq4 = []
            upd = []
            for i in range(4):
                j = grp * 4 + i
                q1 = qq[:, i * HD:(i + 1) * HD]
                k1 = k_ref[rs, j * HD:(j + 1) * HD]
                qk = lax.dot_general(q1, k1, (((1,), (1,)), ((), ())),
                                     preferred_element_type=F32)
                aq4.append(jnp.where(incl, beta * qk, 0.0).astype(BF16))
                upd.append(lax.dot_general(
                    k1, eb4[:, i * HD:(i + 1) * HD], (((0,), (0,)), ((), ())),
                    preferred_element_type=F32))
            aqc = jnp.concatenate(aq4, axis=1)
            ao = jnp.dot(aqc, ed, preferred_element_type=F32)
            o_ref[rs, grp * q4:(grp + 1) * q4] = (
                xqn[r:] + ao).astype(BF16)
            n_ref[grp] = n4 + beta * jnp.concatenate(upd, axis=1)


def kernel(x, Wq, Wk, Wv, Wo):
    b, s, d = x.shape
    r = C * b
    nc = s // C
    beta = LR / b
    hpc = H // 2

    xt = x.transpose(1, 0, 2).reshape(s * b, d).astype(BF16)
    wqkv = jnp.concatenate([Wq.T, Wk.T, Wv.T], axis=1).astype(BF16)

    qkv = _matmul(xt, wqkv, BF16, bm=1024, bn=1024)

    cg_n = 8
    cb_n = 4
    solve = functools.partial(_solve_body, r=r, beta=beta, cg_n=cg_n)
    wx = pl.pallas_call(
        solve,
        grid=(H // 2, nc // cg_n),
        in_specs=[
            pl.BlockSpec((cg_n * r, 2 * HD),
                         lambda p, c: (c, H // 2 + p)),
            pl.BlockSpec((cg_n * r, 2 * HD),
                         lambda p, c: (c, H + p)),
        ],
        out_specs=pl.BlockSpec((cg_n * r, 4 * HD),
                               lambda p, c: (p * (nc // cg_n) + c, 0)),
        out_shape=jax.ShapeDtypeStruct((nc * (H // 2) * r, 4 * HD), BF16),
        compiler_params=pltpu.CompilerParams(
            dimension_semantics=("parallel", "parallel")),
        name="chunk_solve",
    )(qkv, qkv)

    sweep = functools.partial(_sweep_body, r=r, beta=beta, hpc=hpc, cb_n=cb_n)
    nb_c = nc // cb_n
    wx_spec = [
        pl.BlockSpec((cb_n * r, 4 * HD),
                     functools.partial(
                         lambda i, gg, c: ((gg * 4 + i) * nb_c + c, 0), i))
        for i in range(4)
    ]
    o = pl.pallas_call(
        sweep,
        grid=(2, nb_c),
        in_specs=wx_spec + [
            pl.BlockSpec((cb_n * r, hpc * HD), lambda gg, c: (c, gg)),
            pl.BlockSpec((cb_n * r, hpc * HD), lambda gg, c: (c, 2 + gg)),
        ],
        out_specs=pl.BlockSpec((cb_n * r, hpc * HD), lambda gg, c: (c, gg)),
        out_shape=jax.ShapeDtypeStruct((s * b, d), BF16),
        scratch_shapes=[pltpu.VMEM((hpc // 4, HD, 4 * HD), F32)],
        compiler_params=pltpu.CompilerParams(
            dimension_semantics=("parallel", "arbitrary")),
        name="chunk_sweep",
    )(wx, wx, wx, wx, qkv, qkv)

    out = _matmul(o, Wo.T.astype(BF16), F32, bm=1024, bn=1024)
    return out.reshape(s, b, d).transpose(1, 0, 2)


# speedup vs baseline: 2.6459x; 2.0334x over previous
import functools

import jax
import jax.numpy as jnp
from jax import lax
from jax.experimental import pallas as pl
from jax.experimental.pallas import tpu as pltpu

H = 16
HD = 64
LR = 0.1
C = 32
F32 = jnp.float32


BF16 = jnp.bfloat16


def _mm_body(x_ref, w_ref, o_ref):
    o_ref[...] = jnp.dot(x_ref[...], w_ref[...],
                         preferred_element_type=F32).astype(o_ref.dtype)


def _matmul(x, w, out_dtype, bm=1024, bn=1024):
    m, k = x.shape
    _, n = w.shape
    return pl.pallas_call(
        _mm_body,
        grid=(m // bm, n // bn),
        in_specs=[
            pl.BlockSpec((bm, k), lambda i, j: (i, 0)),
            pl.BlockSpec((k, bn), lambda i, j: (0, j)),
        ],
        out_specs=pl.BlockSpec((bm, bn), lambda i, j: (i, j)),
        out_shape=jax.ShapeDtypeStruct((m, n), out_dtype),
        compiler_params=pltpu.CompilerParams(
            dimension_semantics=("parallel", "parallel")),
        name="proj_mm",
    )(x, w)


def _solve_body(k_ref, v_ref, wx_ref, *, r, beta, cg_n):
    r2 = 2 * r
    rows = lax.broadcasted_iota(jnp.int32, (r, r2), 0)
    cols = lax.broadcasted_iota(jnp.int32, (r, r2), 1)
    colm = cols & (r - 1)
    strict_p = (colm >> 2) < (rows >> 2)
    ident2 = jnp.where(colm == rows, 1.0, 0.0).astype(BF16)
    lane_lo = cols < r
    lane_even64 = ((cols >> 6) & 1) == 0
    sb = float(beta) ** 0.5

    def bdiag(tp):
        top = jnp.where(lane_lo, tp, jnp.bfloat16(0))
        bot = jnp.where(lane_lo, jnp.bfloat16(0), tp)
        return jnp.concatenate([top, bot], axis=0)

    m4 = (((rows >> 2) & 1) == 1) & ((colm >> 2) == (rows >> 2) - 1)
    ads, ts = [], []
    for ci in range(cg_n):
        k12 = k_ref[ci * r:(ci + 1) * r, :]
        k1s = k12[:, :HD] * sb
        k2s = k12[:, HD:] * sb
        g1 = lax.dot_general(k1s, k1s, (((1,), (1,)), ((), ())),
                             preferred_element_type=F32)
        g2 = lax.dot_general(k2s, k2s, (((1,), (1,)), ((), ())),
                             preferred_element_type=F32)
        gp = jnp.concatenate([g1, g2], axis=1)
        ads.append(bdiag(jnp.where(strict_p, gp, 0.0).astype(BF16)))
        ts.append(ident2 - jnp.where(m4, gp, 0.0).astype(BF16))
    gsz, sh = 8, 3
    while gsz < r:
        rg = rows >> sh
        cg = colm >> sh
        msub = ((rg & 1) == 1) & (cg == rg - 1)
        for ci in range(cg_n):
            t = ts[ci]
            td = bdiag(t)
            u = jnp.dot(t, ads[ci], preferred_element_type=F32)
            u = jnp.dot(u.astype(BF16), td, preferred_element_type=F32)
            ts[ci] = t - jnp.where(msub, u.astype(BF16), jnp.bfloat16(0))
        gsz, sh = gsz * 2, sh + 1
    for ci in range(cg_n):
        v12 = v_ref[ci * r:(ci + 1) * r, :]
        k12 = k_ref[ci * r:(ci + 1) * r, :]
        vkcat = jnp.concatenate([v12, k12], axis=1)
        rhs = jnp.concatenate([
            jnp.where(lane_even64, vkcat, jnp.bfloat16(0)),
            jnp.where(lane_even64, jnp.bfloat16(0), vkcat),
        ], axis=0)
        wx_ref[ci * r:(ci + 1) * r, :] = jnp.dot(
            ts[ci], rhs, preferred_element_type=F32).astype(BF16)


def _sweep_body(wx0, wx1, wx2, wx3, q_ref, k_ref, o_ref, n_ref,
                *, r, beta, hpc, cb_n):
    c = pl.program_id(1)

    @pl.when(c == 0)
    def _():
        n_ref[...] = jnp.zeros_like(n_ref)

    rows = lax.broadcasted_iota(jnp.int32, (r, r), 0)
    cols = lax.broadcasted_iota(jnp.int32, (r, r), 1)
    incl = (cols >> 2) <= (rows >> 2)
    wxr = (wx0, wx1, wx2, wx3)
    q4 = 4 * HD

    def bdiag4(p4):
        lane = lax.broadcasted_iota(jnp.int32, p4.shape, 1) >> 6
        return jnp.concatenate(
            [jnp.where(lane == i, p4, jnp.bfloat16(0)) for i in range(4)],
            axis=0)

    for cc in range(cb_n):
        rs = slice(cc * r, (cc + 1) * r)
        for grp in range(hpc // 4):
            pr0 = grp * 2
            w4 = jnp.concatenate(
                [wxr[pr0][rs, 0:2 * HD], wxr[pr0 + 1][rs, 0:2 * HD]], axis=1)
            x4 = jnp.concatenate(
                [wxr[pr0][rs, 2 * HD:4 * HD],
                 wxr[pr0 + 1][rs, 2 * HD:4 * HD]], axis=1)
            qq = q_ref[rs, grp * q4:(grp + 1) * q4]
            n4 = n_ref[grp]
            nd = bdiag4(n4.astype(BF16))
            xq = jnp.concatenate([x4, qq], axis=0)
            xqn = jnp.dot(xq, nd, preferred_element_type=F32)
            e4 = w4.astype(F32) - xqn[:r]
            eb4 = e4.astype(BF16)
            ed = bdiag4(eb4)
            aq4 = []
            upd = []
            for i in range(4):
                j = grp * 4 + i
                q1 = qq[:, i * HD:(i + 1) * HD]
                k1 = k_ref[rs, j * HD:(j + 1) * HD]
                qk = lax.dot_general(q1, k1, (((1,), (1,)), ((), ())),
                                     preferred_element_type=F32)
                aq4.append(jnp.where(incl, beta * qk, 0.0).astype(BF16))
                upd.append(lax.dot_general(
                    k1, eb4[:, i * HD:(i + 1) * HD], (((0,), (0,)), ((), ())),
                    preferred_element_type=F32))
            aqc = jnp.concatenate(aq4, axis=1)
            ao = jnp.dot(aqc, ed, preferred_element_type=F32)
            o_ref[rs, grp * q4:(grp + 1) * q4] = (
                xqn[r:] + ao).astype(BF16)
            n_ref[grp] = n4 + beta * jnp.concatenate(upd, axis=1)


def kernel(x, Wq, Wk, Wv, Wo):
    b, s, d = x.shape
    r = C * b
    nc = s // C
    beta = LR / b
    hpc = H // 2

    xt = x.transpose(1, 0, 2).reshape(s * b, d).astype(BF16)
    wqkv = jnp.concatenate([Wq.T, Wk.T, Wv.T], axis=1).astype(BF16)

    qkv = _matmul(xt, wqkv, BF16, bm=1024, bn=1024)

    cg_n = 8
    cb_n = 4
    solve = functools.partial(_solve_body, r=r, beta=beta, cg_n=cg_n)
    wx = pl.pallas_call(
        solve,
        grid=(H // 2, nc // cg_n),
        in_specs=[
            pl.BlockSpec((cg_n * r, 2 * HD),
                         lambda p, c: (c, H // 2 + p)),
            pl.BlockSpec((cg_n * r, 2 * HD),
                         lambda p, c: (c, H + p)),
        ],
        out_specs=pl.BlockSpec((cg_n * r, 4 * HD),
                               lambda p, c: (p * (nc // cg_n) + c, 0)),
        out_shape=jax.ShapeDtypeStruct((nc * (H // 2) * r, 4 * HD), BF16),
        compiler_params=pltpu.CompilerParams(
            dimension_semantics=("parallel", "parallel")),
        name="chunk_solve",
    )(qkv, qkv)

    sweep = functools.partial(_sweep_body, r=r, beta=beta, hpc=hpc, cb_n=cb_n)
    nb_c = nc // cb_n
    wx_spec = [
        pl.BlockSpec((cb_n * r, 4 * HD),
                     functools.partial(
                         lambda i, gg, c: ((gg * 4 + i) * nb_c + c, 0), i))
        for i in range(4)
    ]
    o = pl.pallas_call(
        sweep,
        grid=(2, nb_c),
        in_specs=wx_spec + [
            pl.BlockSpec((cb_n * r, hpc * HD), lambda gg, c: (c, gg)),
            pl.BlockSpec((cb_n * r, hpc * HD), lambda gg, c: (c, 2 + gg)),
        ],
        out_specs=pl.BlockSpec((cb_n * r, hpc * HD), lambda gg, c: (c, gg)),
        out_shape=jax.ShapeDtypeStruct((s * b, d), BF16),
        scratch_shapes=[pltpu.VMEM((hpc // 4, HD, 4 * HD), F32)],
        compiler_params=pltpu.CompilerParams(
            dimension_semantics=("parallel", "arbitrary")),
        name="chunk_sweep",
    )(wx, wx, wx, wx, qkv, qkv)

    out = _matmul(o, Wo.T.astype(BF16), F32, bm=1024, bn=1024)
    return out.reshape(s, b, d).transpose(1, 0, 2)


# speedup vs baseline: 2.7544x; 1.0410x over previous
import functools

import jax
import jax.numpy as jnp
from jax import lax
from jax.experimental import pallas as pl
from jax.experimental.pallas import tpu as pltpu

H = 16
HD = 64
LR = 0.1
C = 32
F32 = jnp.float32


BF16 = jnp.bfloat16


def _mm_body(x_ref, w_ref, o_ref):
    o_ref[...] = jnp.dot(x_ref[...], w_ref[...],
                         preferred_element_type=F32).astype(o_ref.dtype)


def _matmul(x, w, out_dtype, bm=1024, bn=1024):
    m, k = x.shape
    _, n = w.shape
    return pl.pallas_call(
        _mm_body,
        grid=(m // bm, n // bn),
        in_specs=[
            pl.BlockSpec((bm, k), lambda i, j: (i, 0)),
            pl.BlockSpec((k, bn), lambda i, j: (0, j)),
        ],
        out_specs=pl.BlockSpec((bm, bn), lambda i, j: (i, j)),
        out_shape=jax.ShapeDtypeStruct((m, n), out_dtype),
        compiler_params=pltpu.CompilerParams(
            dimension_semantics=("parallel", "parallel")),
        name="proj_mm",
    )(x, w)


def _solve_body(k_ref, v_ref, wx_ref, *, r, beta, cg_n):
    r2 = 2 * r
    rows = lax.broadcasted_iota(jnp.int32, (r, r2), 0)
    cols = lax.broadcasted_iota(jnp.int32, (r, r2), 1)
    colm = cols & (r - 1)
    strict_p = (colm >> 2) < (rows >> 2)
    ident2 = jnp.where(colm == rows, 1.0, 0.0).astype(BF16)
    lane_lo = cols < r
    lane_even64 = ((cols >> 6) & 1) == 0
    sb = float(beta) ** 0.5

    def bdiag(tp):
        top = jnp.where(lane_lo, tp, jnp.bfloat16(0))
        bot = jnp.where(lane_lo, jnp.bfloat16(0), tp)
        return jnp.concatenate([top, bot], axis=0)

    m4 = (((rows >> 2) & 1) == 1) & ((colm >> 2) == (rows >> 2) - 1)
    ads, ts = [], []
    for ci in range(cg_n):
        k12 = k_ref[ci * r:(ci + 1) * r, :]
        k1s = k12[:, :HD] * sb
        k2s = k12[:, HD:] * sb
        g1 = lax.dot_general(k1s, k1s, (((1,), (1,)), ((), ())),
                             preferred_element_type=F32)
        g2 = lax.dot_general(k2s, k2s, (((1,), (1,)), ((), ())),
                             preferred_element_type=F32)
        gp = jnp.concatenate([g1, g2], axis=1)
        ads.append(bdiag(jnp.where(strict_p, gp, 0.0).astype(BF16)))
        ts.append(ident2 - jnp.where(m4, gp, 0.0).astype(BF16))
    gsz, sh = 8, 3
    while gsz < r:
        rg = rows >> sh
        cg = colm >> sh
        msub = ((rg & 1) == 1) & (cg == rg - 1)
        for ci in range(cg_n):
            t = ts[ci]
            td = bdiag(t)
            u = jnp.dot(t, ads[ci], preferred_element_type=F32)
            u = jnp.dot(u.astype(BF16), td, preferred_element_type=F32)
            ts[ci] = t - jnp.where(msub, u.astype(BF16), jnp.bfloat16(0))
        gsz, sh = gsz * 2, sh + 1
    for ci in range(cg_n):
        v12 = v_ref[ci * r:(ci + 1) * r, :]
        k12 = k_ref[ci * r:(ci + 1) * r, :]
        vkcat = jnp.concatenate([v12, k12], axis=1)
        rhs = jnp.concatenate([
            jnp.where(lane_even64, vkcat, jnp.bfloat16(0)),
            jnp.where(lane_even64, jnp.bfloat16(0), vkcat),
        ], axis=0)
        wx_ref[ci * r:(ci + 1) * r, :] = jnp.dot(
            ts[ci], rhs, preferred_element_type=F32).astype(BF16)


def _sweep_body(wx0, wx1, wx2, wx3, q_ref, k_ref, o_ref, n_ref,
                *, r, beta, hpc, cb_n):
    c = pl.program_id(1)

    @pl.when(c == 0)
    def _():
        n_ref[...] = jnp.zeros_like(n_ref)

    rows = lax.broadcasted_iota(jnp.int32, (r, r), 0)
    cols = lax.broadcasted_iota(jnp.int32, (r, r), 1)
    incl = (cols >> 2) <= (rows >> 2)
    wxr = (wx0, wx1, wx2, wx3)
    q4 = 4 * HD

    def bdiag4(p4):
        lane = lax.broadcasted_iota(jnp.int32, p4.shape, 1) >> 6
        return jnp.concatenate(
            [jnp.where(lane == i, p4, jnp.bfloat16(0)) for i in range(4)],
            axis=0)

    for cc in range(cb_n):
        rs = slice(cc * r, (cc + 1) * r)
        for grp in range(hpc // 4):
            pr0 = grp * 2
            w4 = jnp.concatenate(
                [wxr[pr0][rs, 0:2 * HD], wxr[pr0 + 1][rs, 0:2 * HD]], axis=1)
            x4 = jnp.concatenate(
                [wxr[pr0][rs, 2 * HD:4 * HD],
                 wxr[pr0 + 1][rs, 2 * HD:4 * HD]], axis=1)
            qq = q_ref[rs, grp * q4:(grp + 1) * q4]
            n4 = n_ref[grp]
            nd = bdiag4(n4.astype(BF16))
            xq = jnp.concatenate([x4, qq], axis=0)
            xqn = jnp.dot(xq, nd, preferred_element_type=F32)
            e4 = w4.astype(F32) - xqn[:r]
            eb4 = e4.astype(BF16)
            ed = bdiag4(eb4)
            aq4 = []
            upd = []
            for i in range(4):
                j = grp * 4 + i
                q1 = qq[:, i * HD:(i + 1) * HD]
                k1 = k_ref[rs, j * HD:(j + 1) * HD]
                qk = lax.dot_general(q1, k1, (((1,), (1,)), ((), ())),
                                     preferred_element_type=F32)
                aq4.append(jnp.where(incl, beta * qk, 0.0).astype(BF16))
                upd.append(lax.dot_general(
                    k1, eb4[:, i * HD:(i + 1) * HD], (((0,), (0,)), ((), ())),
                    preferred_element_type=F32))
            aqc = jnp.concatenate(aq4, axis=1)
            ao = jnp.dot(aqc, ed, preferred_element_type=F32)
            o_ref[rs, grp * q4:(grp + 1) * q4] = (
                xqn[r:] + ao).astype(BF16)
            n_ref[grp] = n4 + beta * jnp.concatenate(upd, axis=1)


def kernel(x, Wq, Wk, Wv, Wo):
    b, s, d = x.shape
    r = C * b
    nc = s // C
    beta = LR / b
    hpc = H // 2

    xt = x.transpose(1, 0, 2).reshape(s * b, d).astype(BF16)
    wqkv = jnp.concatenate([Wq.T, Wk.T, Wv.T], axis=1).astype(BF16)

    qkv = _matmul(xt, wqkv, BF16, bm=1024, bn=1024)

    cg_n = 16
    cb_n = 8
    solve = functools.partial(_solve_body, r=r, beta=beta, cg_n=cg_n)
    wx = pl.pallas_call(
        solve,
        grid=(H // 2, nc // cg_n),
        in_specs=[
            pl.BlockSpec((cg_n * r, 2 * HD),
                         lambda p, c: (c, H // 2 + p)),
            pl.BlockSpec((cg_n * r, 2 * HD),
                         lambda p, c: (c, H + p)),
        ],
        out_specs=pl.BlockSpec((cg_n * r, 4 * HD),
                               lambda p, c: (p * (nc // cg_n) + c, 0)),
        out_shape=jax.ShapeDtypeStruct((nc * (H // 2) * r, 4 * HD), BF16),
        compiler_params=pltpu.CompilerParams(
            dimension_semantics=("parallel", "parallel")),
        name="chunk_solve",
    )(qkv, qkv)

    sweep = functools.partial(_sweep_body, r=r, beta=beta, hpc=hpc, cb_n=cb_n)
    nb_c = nc // cb_n
    wx_spec = [
        pl.BlockSpec((cb_n * r, 4 * HD),
                     functools.partial(
                         lambda i, gg, c: ((gg * 4 + i) * nb_c + c, 0), i))
        for i in range(4)
    ]
    o = pl.pallas_call(
        sweep,
        grid=(2, nb_c),
        in_specs=wx_spec + [
            pl.BlockSpec((cb_n * r, hpc * HD), lambda gg, c: (c, gg)),
            pl.BlockSpec((cb_n * r, hpc * HD), lambda gg, c: (c, 2 + gg)),
        ],
        out_specs=pl.BlockSpec((cb_n * r, hpc * HD), lambda gg, c: (c, gg)),
        out_shape=jax.ShapeDtypeStruct((s * b, d), BF16),
        scratch_shapes=[pltpu.VMEM((hpc // 4, HD, 4 * HD), F32)],
        compiler_params=pltpu.CompilerParams(
            dimension_semantics=("parallel", "arbitrary")),
        name="chunk_sweep",
    )(wx, wx, wx, wx, qkv, qkv)

    out = _matmul(o, Wo.T.astype(BF16), F32, bm=1024, bn=1024)
    return out.reshape(s, b, d).transpose(1, 0, 2)
